# Initial kernel scaffold; baseline (speedup 1.0000x reference)
import jax
import jax.numpy as jnp
from jax.experimental import pallas as pl


def kernel(wd, bd, g1, be1, w3, b3, g2, be2, wp, bp, x):
    raise NotImplementedError("write your pallas kernel here")


# speedup vs baseline: 1.9962x; 1.9962x over previous
import functools

import jax
import jax.numpy as jnp
from jax import lax
from jax.experimental import pallas as pl
from jax.experimental.pallas import tpu as pltpu

_VMEM_LIMIT = 48 * 1024 * 1024


def _cparams(n_axes):
    return pltpu.CompilerParams(
        dimension_semantics=("parallel",) * n_axes,
        vmem_limit_bytes=_VMEM_LIMIT,
    )


def _gelu(x):
    c = 0.7978845608028654
    return 0.5 * x * (1.0 + jnp.tanh(c * (x + 0.044715 * x * x * x)))


def _gram_kernel(x_ref, o_ref):
    _, cin, h, w = x_ref.shape
    xb = x_ref[...].reshape(cin, h * w)
    xb1 = jnp.concatenate([xb, jnp.ones((1, h * w), jnp.float32)], axis=0)
    g = lax.dot_general(xb1, xb1, (((1,), (1,)), ((), ())),
                        preferred_element_type=jnp.float32)
    o_ref[...] = g.reshape(1, cin + 1, cin + 1)


def _fused_mid_kernel(nH, cmid, top_ref, main_ref, bot_ref, w00_ref, w01_ref,
                      w10_ref, w11_ref, bd_ref, sc1_ref, sh1_ref, w3_ref,
                      b3_ref, z_ref, st_ref):
    i = pl.program_id(0)
    h = i % nH
    _, rin, w, cin = main_ref.shape
    trh = 2 * rin
    ow = 2 * w
    nr = rin + 2
    xa = jnp.concatenate([top_ref[0], main_ref[0], bot_ref[0]], axis=0)
    xa = xa.reshape(nr * w, cin)
    bd = bd_ref[...]
    sc1 = sc1_ref[...]
    sh1 = sh1_ref[...]

    def tap(w_ref):
        y = jnp.dot(xa, w_ref[...], preferred_element_type=jnp.float32)
        return _gelu((y + bd) * sc1 + sh1).reshape(nr, w, cmid)

    a00, a01, a10, a11 = tap(w00_ref), tap(w01_ref), tap(w10_ref), tap(w11_ref)
    r0 = jnp.stack([a00, a01], axis=2).reshape(nr, ow, cmid)
    r1 = jnp.stack([a10, a11], axis=2).reshape(nr, ow, cmid)
    aimg = jnp.stack([r0, r1], axis=1).reshape(2 * nr, ow, cmid)
    sel = aimg[1: trh + 3]
    r_idx = lax.broadcasted_iota(jnp.int32, (trh + 2, 1, 1), 0)
    kill = jnp.logical_or(jnp.logical_and(h == 0, r_idx == 0),
                          jnp.logical_and(h == nH - 1, r_idx == trh + 1))
    sel = jnp.where(kill, 0.0, sel)
    cat = jnp.pad(sel, ((0, 0), (1, 1), (0, 0)))
    acc = jnp.zeros((trh * ow, cmid), jnp.float32)
    for kh in range(3):
        for kw in range(3):
            patch = cat[kh:kh + trh, kw:kw + ow, :].reshape(trh * ow, cmid)
            w_tap = w3_ref[(kh * 3 + kw) * cmid:(kh * 3 + kw + 1) * cmid, :]
            acc = acc + jnp.dot(patch, w_tap,
                                preferred_element_type=jnp.float32)
    z = acc + b3_ref[...]
    z_ref[...] = z.reshape(1, trh, ow, cmid)
    s = jnp.sum(z, axis=0, keepdims=True)
    ss = jnp.sum(z * z, axis=0, keepdims=True)
    st_ref[...] = jnp.concatenate([s, ss], axis=0).reshape(1, 2, cmid)


def _head_kernel(nc, z_ref, sc_ref, sh_ref, wpt_ref, bp_ref, o_ref):
    _, tr, ow, cmid = z_ref.shape
    m = tr * ow
    zb = z_ref[...].reshape(m, cmid)
    a = _gelu(zb * sc_ref[...] + sh_ref[...])
    lt = lax.dot_general(wpt_ref[...], a, (((1,), (1,)), ((), ())),
                         preferred_element_type=jnp.float32)
    lt = lt + bp_ref[...]
    o_ref[...] = lt[:nc].reshape(1, nc, tr, ow)


def kernel(wd, bd, g1, be1, w3, b3, g2, be2, wp, bp, x):
    f32 = jnp.float32
    eps = 1e-5
    B, Cin, H, W = x.shape
    Cmid = wd.shape[1]
    NC = wp.shape[0]
    OH, OW = 2 * H, 2 * W
    M0 = B * H * W
    M1 = B * OH * OW
    x = x.astype(f32)

    gout = pl.pallas_call(
        _gram_kernel,
        out_shape=jax.ShapeDtypeStruct((B, Cin + 1, Cin + 1), f32),
        grid=(B,),
        in_specs=[pl.BlockSpec((1, Cin, H, W), lambda b: (b, 0, 0, 0))],
        out_specs=pl.BlockSpec((1, Cin + 1, Cin + 1), lambda b: (b, 0, 0)),
        compiler_params=_cparams(1),
    )(x)
    gtot = jnp.sum(gout, axis=0)
    G = gtot[:Cin, :Cin]
    s = gtot[Cin, :Cin]

    wdp = jnp.transpose(wd.astype(f32), (0, 2, 3, 1))
    wdf = wdp.reshape(Cin, 4 * Cmid)
    bd4 = jnp.tile(bd.astype(f32), (4,))
    sv = s @ wdf
    qv = jnp.sum(wdf * (G @ wdf), axis=0)
    sum_col = sv + M0 * bd4
    sumsq_col = qv + 2.0 * bd4 * sv + M0 * bd4 * bd4
    mean1 = jnp.sum(sum_col.reshape(4, Cmid), axis=0) / M1
    e2 = jnp.sum(sumsq_col.reshape(4, Cmid), axis=0) / M1
    var1 = e2 - mean1 * mean1
    scale1 = g1.astype(f32) * lax.rsqrt(var1 + eps)
    shift1 = be1.astype(f32) - mean1 * scale1
    sc1 = scale1.reshape(1, Cmid)
    sh1 = shift1.reshape(1, Cmid)
    bdr = bd.astype(f32).reshape(1, Cmid)

    x_img = jnp.transpose(x, (0, 2, 3, 1))
    RIN = 8
    nH = H // RIN
    TRH = 2 * RIN
    w00 = wdp[:, 0, 0]
    w01 = wdp[:, 0, 1]
    w10 = wdp[:, 1, 0]
    w11 = wdp[:, 1, 1]
    w3s = jnp.transpose(w3.astype(f32), (2, 3, 1, 0)).reshape(9 * Cmid, Cmid)
    b3r = b3.astype(f32).reshape(1, Cmid)
    grid2 = B * nH
    wspec = pl.BlockSpec((Cin, Cmid), lambda i: (0, 0))
    vspec = pl.BlockSpec((1, Cmid), lambda i: (0, 0))
    z, st = pl.pallas_call(
        functools.partial(_fused_mid_kernel, nH, Cmid),
        out_shape=(jax.ShapeDtypeStruct((B, OH, OW, Cmid), f32),
                   jax.ShapeDtypeStruct((grid2, 2, Cmid), f32)),
        grid=(grid2,),
        in_specs=[
            pl.BlockSpec((1, 1, W, Cin),
                         lambda i: (i // nH, jnp.maximum((i % nH) * RIN - 1, 0), 0, 0)),
            pl.BlockSpec((1, RIN, W, Cin), lambda i: (i // nH, i % nH, 0, 0)),
            pl.BlockSpec((1, 1, W, Cin),
                         lambda i: (i // nH, jnp.minimum((i % nH) * RIN + RIN, H - 1), 0, 0)),
            wspec, wspec, wspec, wspec,
            vspec, vspec, vspec,
            pl.BlockSpec((9 * Cmid, Cmid), lambda i: (0, 0)),
            vspec,
        ],
        out_specs=(
            pl.BlockSpec((1, TRH, OW, Cmid), lambda i: (i // nH, i % nH, 0, 0)),
            pl.BlockSpec((1, 2, Cmid), lambda i: (i, 0, 0)),
        ),
        compiler_params=_cparams(1),
    )(x_img, x_img, x_img, w00, w01, w10, w11, bdr, sc1, sh1, w3s, b3r)

    tot2 = jnp.sum(st, axis=0)
    mean2 = tot2[0] / M1
    var2 = tot2[1] / M1 - mean2 * mean2
    scale2 = g2.astype(f32) * lax.rsqrt(var2 + eps)
    shift2 = be2.astype(f32) - mean2 * scale2
    sc2 = scale2.reshape(1, Cmid)
    sh2 = shift2.reshape(1, Cmid)

    NCP = ((NC + 7) // 8) * 8
    wpt = jnp.zeros((NCP, Cmid), f32).at[:NC].set(wp.astype(f32)[:, :, 0, 0])
    bpc = jnp.zeros((NCP, 1), f32).at[:NC, 0].set(bp.astype(f32))
    TR3 = 32
    nH3 = OH // TR3
    grid3 = B * nH3
    logits = pl.pallas_call(
        functools.partial(_head_kernel, NC),
        out_shape=jax.ShapeDtypeStruct((B, NC, OH, OW), f32),
        grid=(grid3,),
        in_specs=[
            pl.BlockSpec((1, TR3, OW, Cmid), lambda i: (i // nH3, i % nH3, 0, 0)),
            pl.BlockSpec((1, Cmid), lambda i: (0, 0)),
            pl.BlockSpec((1, Cmid), lambda i: (0, 0)),
            pl.BlockSpec((NCP, Cmid), lambda i: (0, 0)),
            pl.BlockSpec((NCP, 1), lambda i: (0, 0)),
        ],
        out_specs=pl.BlockSpec((1, NC, TR3, OW), lambda i: (i // nH3, 0, i % nH3, 0)),
        compiler_params=_cparams(1),
    )(z, sc2, sh2, wpt, bpc)
    return logits
